# baseline (speedup 1.0000x reference)
import functools

import jax
import jax.numpy as jnp
from jax import lax
from jax.experimental import pallas as pl
from jax.experimental.pallas import tpu as pltpu
from jax.experimental.pallas import tpu_sc as plsc

_NC = 2
_NS = 16
_NW = _NC * _NS

_CHUNK = 128
_NBUF = 8
_LAG = 4


@jax.jit
def _embedding_sc(idx_flat, weight):
    n = idx_flat.shape[0]
    _, d = weight.shape
    per_w = n // _NW
    n_chunks = per_w // _CHUNK
    assert n == per_w * _NW and per_w == n_chunks * _CHUNK
    assert n_chunks % _NBUF == 0 and n_chunks >= 2 * _NBUF

    mesh = plsc.VectorSubcoreMesh(core_axis_name="c", subcore_axis_name="s")

    @functools.partial(
        pl.kernel,
        mesh=mesh,
        compiler_params=pltpu.CompilerParams(use_tc_tiling_on_sc=False),
        out_type=jax.ShapeDtypeStruct((n, d), jnp.bfloat16),
        scratch_types=[
            pltpu.VMEM((per_w,), jnp.int32),
            pltpu.VMEM((_NBUF, _CHUNK, d), jnp.bfloat16),
            pltpu.SemaphoreType.DMA,
            pltpu.SemaphoreType.DMA,
        ],
    )
    def emb(idx_hbm, table_hbm, out_hbm, idx_v, rows_v, gsem, osem):
        wid = lax.axis_index("s") * _NC + lax.axis_index("c")
        base = wid * per_w
        pltpu.sync_copy(idx_hbm.at[pl.ds(base, per_w)], idx_v)

        def fire_gather(c, buf):
            pltpu.async_copy(
                table_hbm.at[idx_v.at[pl.ds(c * _CHUNK, _CHUNK)]],
                rows_v.at[buf],
                gsem,
            )

        def wait_gather(buf):
            pltpu.make_async_copy(
                table_hbm.at[pl.ds(0, _CHUNK)], rows_v.at[buf], gsem
            ).wait()

        def fire_write(c, buf):
            pltpu.async_copy(
                rows_v.at[buf], out_hbm.at[pl.ds(base + c * _CHUNK, _CHUNK)], osem
            )

        def wait_write(buf):
            pltpu.make_async_copy(
                table_hbm.at[pl.ds(0, _CHUNK)], rows_v.at[buf], osem
            ).wait()

        for b in range(_LAG):
            fire_gather(b, b)
        for c in range(_LAG):
            wait_gather(c % _NBUF)
            fire_write(c, c % _NBUF)
            fire_gather(c + _LAG, (c + _LAG) % _NBUF)

        def steady(c0, _):
            for b in range(_NBUF):
                bw = (_LAG + b) % _NBUF
                wait_gather(bw)
                fire_write(c0 + b, bw)
                wait_write((bw + _LAG) % _NBUF)
                fire_gather(c0 + b + _LAG, (bw + _LAG) % _NBUF)
            return ()

        lax.fori_loop(
            0, (n_chunks - 2 * _LAG) // _NBUF,
            lambda i, c: steady(_LAG + i * _NBUF, c), (),
        )

        for b in range(_LAG):
            c = n_chunks - _LAG + b
            bw = c % _NBUF
            wait_gather(bw)
            fire_write(c, bw)
            wait_write((bw + _LAG) % _NBUF)
        for b in range(_LAG):
            bw = (n_chunks - _LAG + b) % _NBUF
            wait_write(bw)

    return emb(idx_flat, weight)


def kernel(x, weight):
    b, hist = x.shape
    _, d = weight.shape
    out = _embedding_sc(x.reshape(b * hist), weight.astype(jnp.bfloat16))
    return out.astype(jnp.float32).reshape(b, hist, d)


# speedup vs baseline: 1.4728x; 1.4728x over previous
import functools

import jax
import jax.numpy as jnp
from jax import lax
from jax.experimental import pallas as pl
from jax.experimental.pallas import tpu as pltpu
from jax.experimental.pallas import tpu_sc as plsc

_NC = 2
_NS = 16
_NW = _NC * _NS

_CHUNK = 128
_NBUF = 8
_LAG = 4


@jax.jit
def _embedding_sc(idx_flat, weight):
    n = idx_flat.shape[0]
    _, d = weight.shape
    per_w = n // _NW
    n_chunks = per_w // _CHUNK
    assert n == per_w * _NW and per_w == n_chunks * _CHUNK
    assert n_chunks % _NBUF == 0 and n_chunks >= 2 * _NBUF

    mesh = plsc.VectorSubcoreMesh(core_axis_name="c", subcore_axis_name="s")

    @functools.partial(
        pl.kernel,
        mesh=mesh,
        compiler_params=pltpu.CompilerParams(use_tc_tiling_on_sc=False),
        out_type=jax.ShapeDtypeStruct((n, d), jnp.float32),
        scratch_types=[
            pltpu.VMEM((per_w,), jnp.int32),
            pltpu.VMEM((_NBUF, _CHUNK, d), jnp.float32),
            pltpu.SemaphoreType.DMA,
            pltpu.SemaphoreType.DMA,
        ],
    )
    def emb(idx_hbm, table_hbm, out_hbm, idx_v, rows_v, gsem, osem):
        wid = lax.axis_index("s") * _NC + lax.axis_index("c")
        base = wid * per_w
        pltpu.sync_copy(idx_hbm.at[pl.ds(base, per_w)], idx_v)

        def fire_gather(c, buf):
            pltpu.async_copy(
                table_hbm.at[idx_v.at[pl.ds(c * _CHUNK, _CHUNK)]],
                rows_v.at[buf],
                gsem,
            )

        def wait_gather(buf):
            pltpu.make_async_copy(
                table_hbm.at[pl.ds(0, _CHUNK)], rows_v.at[buf], gsem
            ).wait()

        def fire_write(c, buf):
            pltpu.async_copy(
                rows_v.at[buf], out_hbm.at[pl.ds(base + c * _CHUNK, _CHUNK)], osem
            )

        def wait_write(buf):
            pltpu.make_async_copy(
                table_hbm.at[pl.ds(0, _CHUNK)], rows_v.at[buf], osem
            ).wait()

        for b in range(_LAG):
            fire_gather(b, b)
        for c in range(_LAG):
            wait_gather(c % _NBUF)
            fire_write(c, c % _NBUF)
            fire_gather(c + _LAG, (c + _LAG) % _NBUF)

        def steady(c0, _):
            for b in range(_NBUF):
                bw = (_LAG + b) % _NBUF
                wait_gather(bw)
                fire_write(c0 + b, bw)
                wait_write((bw + _LAG) % _NBUF)
                fire_gather(c0 + b + _LAG, (bw + _LAG) % _NBUF)
            return ()

        lax.fori_loop(
            0, (n_chunks - 2 * _LAG) // _NBUF,
            lambda i, c: steady(_LAG + i * _NBUF, c), (),
        )

        for b in range(_LAG):
            c = n_chunks - _LAG + b
            bw = c % _NBUF
            wait_gather(bw)
            fire_write(c, bw)
            wait_write((bw + _LAG) % _NBUF)
        for b in range(_LAG):
            bw = (n_chunks - _LAG + b) % _NBUF
            wait_write(bw)

    return emb(idx_flat, weight)


def kernel(x, weight):
    b, hist = x.shape
    _, d = weight.shape
    out = _embedding_sc(x.reshape(b * hist), weight)
    return out.reshape(b, hist, d)
